# Initial kernel scaffold; baseline (speedup 1.0000x reference)
import jax
import jax.numpy as jnp
from jax.experimental import pallas as pl


def kernel(cost_matrix, rand):
    raise NotImplementedError("write your pallas kernel here")


# speedup vs baseline: 5.4308x; 5.4308x over previous
import jax
import jax.numpy as jnp
from jax.experimental import pallas as pl
from jax.experimental.pallas import tpu as pltpu

BB = 8


def _onehot_body(rand_ref, col_ref, row_ref):
    r = rand_ref[...]
    bb, c = r.shape
    rj = r[:, :, None]
    rk = r[:, None, :]
    jlt = jax.lax.broadcasted_iota(jnp.int32, (bb, c, c), 1) < \
        jax.lax.broadcasted_iota(jnp.int32, (bb, c, c), 2)
    lt_f = jnp.where(rj < rk, 1.0, 0.0)
    tie_f = jnp.where((rj == rk) & jlt, 1.0, 0.0)
    rank = jnp.sum(lt_f + tie_f, axis=1).astype(jnp.int32)
    n_iota = jax.lax.broadcasted_iota(jnp.int32, (bb, c, c), 1)
    col_ref[...] = jnp.where(n_iota == rank[:, None, :], 1.0, 0.0).astype(
        col_ref.dtype)
    row_ref[...] = jnp.zeros(row_ref.shape, row_ref.dtype)


def kernel(cost_matrix, rand):
    b, r, c = cost_matrix.shape
    embed_dim = 256
    grid = (b // BB,)
    col_emb, row_emb = pl.pallas_call(
        _onehot_body,
        grid=grid,
        in_specs=[pl.BlockSpec((BB, c), lambda i: (i, 0))],
        out_specs=[
            pl.BlockSpec((BB, c, embed_dim), lambda i: (i, 0, 0)),
            pl.BlockSpec((BB, r, embed_dim), lambda i: (i, 0, 0)),
        ],
        out_shape=[
            jax.ShapeDtypeStruct((b, c, embed_dim), cost_matrix.dtype),
            jax.ShapeDtypeStruct((b, r, embed_dim), cost_matrix.dtype),
        ],
    )(rand)
    return (row_emb, col_emb, cost_matrix)
